# baseline (speedup 1.0000x reference)
import functools

import jax
import jax.numpy as jnp
from jax import lax
from jax.experimental import pallas as pl
from jax.experimental.pallas import tpu as pltpu
from jax.experimental.pallas import tpu_sc as plsc

N = 50000
E = 800000
NP = 50176
EP = 802816
ER2 = EP // 128
RPT = NP // 16
CONV_ROWS_PT = ER2 // 16
DEG_ROWS_PT = ER2 // 32
PK = NP // 8
SCE = 896
EPT = EP // 16
NB = EPT // (2 * SCE)
DEPC = EP // 32
DNB = DEPC // (2 * SCE)

_mesh = plsc.VectorSubcoreMesh(core_axis_name="c", subcore_axis_name="s")
_sc_params = pltpu.CompilerParams(use_tc_tiling_on_sc=False)


@functools.partial(
    pl.kernel,
    mesh=_mesh,
    compiler_params=_sc_params,
    out_type=(jax.ShapeDtypeStruct((NP, 16), jnp.float32),
              jax.ShapeDtypeStruct((NP, 16), jnp.float32)),
    scratch_types=[
        pltpu.VMEM((SCE,), jnp.int32),
        pltpu.VMEM((SCE,), jnp.int32),
        pltpu.VMEM((SCE, 16), jnp.float32),
        pltpu.VMEM((RPT, 16), jnp.float32),
        pltpu.VMEM_SHARED((NP, 16), jnp.float32),
        pltpu.SemaphoreType.DMA,
        pltpu.SemaphoreType.DMA,
    ],
)
def _deg_kernel(dst1d, zeros_hbm, ones_hbm, deg0, deg1,
                didxa, didxb, ones_v, stage, acc, sema, semb):
    c = lax.axis_index("c")
    s = lax.axis_index("s")
    pltpu.sync_copy(ones_hbm, ones_v)
    pltpu.sync_copy(zeros_hbm.at[pl.ds(s * RPT, RPT)], stage)
    pltpu.sync_copy(stage, acc.at[pl.ds(s * RPT, RPT)])
    plsc.subcore_barrier()
    base = (c * 16 + s) * DEPC

    def drain(sem):
        pltpu.make_async_copy(zeros_hbm.at[pl.ds(0, SCE)], ones_v, sem).wait()

    pltpu.sync_copy(dst1d.at[pl.ds(base, SCE)], didxa)

    def body(i, carry):
        pltpu.async_copy(ones_v, acc.at[didxa], sema, add=True)
        pltpu.sync_copy(dst1d.at[pl.ds(base + (2 * i + 1) * SCE, SCE)], didxb)
        drain(sema)
        pltpu.async_copy(ones_v, acc.at[didxb], semb, add=True)
        rb2 = jnp.minimum(base + (2 * i + 2) * SCE,
                          base + (2 * DNB - 1) * SCE)
        pltpu.sync_copy(dst1d.at[pl.ds(rb2, SCE)], didxa)
        drain(semb)
        return carry

    lax.fori_loop(0, DNB, body, 0)
    plsc.subcore_barrier()

    @pl.when(c == 0)
    def _():
        pltpu.sync_copy(acc.at[pl.ds(s * RPT, RPT)], stage)
        pltpu.sync_copy(stage, deg0.at[pl.ds(s * RPT, RPT)])

    @pl.when(c == 1)
    def _():
        pltpu.sync_copy(acc.at[pl.ds(s * RPT, RPT)], stage)
        pltpu.sync_copy(stage, deg1.at[pl.ds(s * RPT, RPT)])


@functools.partial(
    pl.kernel,
    mesh=_mesh,
    compiler_params=_sc_params,
    out_type=tuple(jax.ShapeDtypeStruct((NP, 16), jnp.float32)
                   for _ in range(4)),
    scratch_types=[
        pltpu.VMEM((SCE,), jnp.int32),
        pltpu.VMEM((SCE,), jnp.int32),
        pltpu.VMEM((SCE,), jnp.int32),
        pltpu.VMEM((SCE,), jnp.int32),
        pltpu.VMEM((SCE, 16), jnp.float32),
        pltpu.VMEM((SCE, 16), jnp.float32),
        pltpu.VMEM_SHARED((NP, 16), jnp.float32),
        pltpu.SemaphoreType.DMA,
        pltpu.SemaphoreType.DMA,
        pltpu.SemaphoreType.DMA,
        pltpu.SemaphoreType.DMA,
    ],
)
def _scatter_kernel(src1d, dst1d, g0, g1, g2, g3, o0, o1, o2, o3,
                    sidx0, sidx1, didx0, didx1, rows0, rows1, acc,
                    gsem0, gsem1, ssem0, ssem1):
    c = lax.axis_index("c")
    s = lax.axis_index("s")

    def run(g_hbm, out_hbm):
        for k in range(4):
            pltpu.sync_copy(g_hbm.at[pl.ds(s * RPT + k * 784, 784)],
                            rows0.at[pl.ds(0, 784)])
            pltpu.sync_copy(rows0.at[pl.ds(0, 784)],
                            acc.at[pl.ds(s * RPT + k * 784, 784)])
        plsc.subcore_barrier()

        base = s * EPT

        def load_idx(eb, sidx, didx):
            pltpu.sync_copy(src1d.at[pl.ds(eb, SCE)], sidx)
            pltpu.sync_copy(dst1d.at[pl.ds(eb, SCE)], didx)

        def drain(sem):
            pltpu.make_async_copy(g_hbm.at[pl.ds(0, SCE)], rows0, sem).wait()

        load_idx(base, sidx0, didx0)
        pltpu.async_copy(g_hbm.at[sidx0], rows0, gsem0)

        def body(i, carry):
            load_idx(base + (2 * i + 1) * SCE, sidx1, didx1)
            drain(gsem0)
            pltpu.async_copy(rows0, acc.at[didx0], ssem0, add=True)
            pltpu.async_copy(g_hbm.at[sidx1], rows1, gsem1)
            drain(ssem0)
            eb2 = jnp.minimum(base + (2 * i + 2) * SCE,
                              base + (2 * NB - 1) * SCE)
            load_idx(eb2, sidx0, didx0)
            drain(gsem1)
            pltpu.async_copy(rows1, acc.at[didx1], ssem1, add=True)

            @pl.when(i < NB - 1)
            def _():
                pltpu.async_copy(g_hbm.at[sidx0], rows0, gsem0)
            drain(ssem1)
            return carry

        lax.fori_loop(0, NB, body, 0)
        plsc.subcore_barrier()
        for k in range(4):
            pltpu.sync_copy(acc.at[pl.ds(s * RPT + k * 784, 784)],
                            rows0.at[pl.ds(0, 784)])
            pltpu.sync_copy(rows0.at[pl.ds(0, 784)],
                            out_hbm.at[pl.ds(s * RPT + k * 784, 784)])

    gs = (g0, g1, g2, g3)
    outs = (o0, o1, o2, o3)
    for q in range(4):
        @pl.when(c == q // 2)
        def _(q=q):
            run(gs[q], outs[q])


_PBLK = 128
_GRID = PK // _PBLK


def _packspec():
    return pl.BlockSpec((_PBLK, 128), lambda i: (i, 0))


def _dinv_pk(d0_ref, d1_ref):
    return lax.rsqrt(d0_ref[...] + d1_ref[...] + 1.0)


def _dinv64(dpk):
    return jnp.concatenate(
        [jnp.broadcast_to(dpk[:, 16 * k:16 * k + 1], (_PBLK, 64))
         for k in range(8)], axis=1)


def _to64(chunks):
    cols = []
    for k in range(8):
        for q in range(4):
            cols.append(chunks[q][:, 16 * k:16 * k + 16])
    return jnp.concatenate(cols, axis=1)


def _from64(o, q):
    return jnp.concatenate(
        [o[:, 64 * k + 16 * q:64 * k + 16 * q + 16] for k in range(8)],
        axis=1)


def _dense1_body(d0, d1, xp, w1, g0, g1, g2, g3, dvo):
    d64 = _dinv64(_dinv_pk(d0, d1))
    xb = xp[...]
    o = jnp.concatenate(
        [jnp.dot(xb[:, 128 * k:128 * k + 128], w1[...],
                 preferred_element_type=jnp.float32) for k in range(8)],
        axis=1) * d64
    outs = (g0, g1, g2, g3)
    for q in range(4):
        outs[q][...] = _from64(o, q)
    dvo[...] = _dinv_pk(d0, d1)


_chunk_out = tuple(jax.ShapeDtypeStruct((PK, 128), jnp.float32)
                   for _ in range(4))

_dense1 = pl.pallas_call(
    _dense1_body,
    grid=(_GRID,),
    in_specs=[_packspec(), _packspec(),
              pl.BlockSpec((_PBLK, 1024), lambda i: (i, 0)),
              pl.BlockSpec((128, 64), lambda i: (0, 0))],
    out_specs=[_packspec()] * 5,
    out_shape=_chunk_out + (jax.ShapeDtypeStruct((PK, 128), jnp.float32),),
)


def _dense2_body(s0, s1, s2, s3, dv, w2, b1p, g20, g21, g22, g23):
    d64 = _dinv64(dv[...])
    h = jnp.maximum(_to64([s0[...], s1[...], s2[...], s3[...]]) * d64
                    + b1p[...], 0.0)
    o = jnp.concatenate(
        [jnp.dot(h[:, 64 * k:64 * k + 64], w2[...],
                 preferred_element_type=jnp.float32) for k in range(8)],
        axis=1) * d64
    outs = (g20, g21, g22, g23)
    for q in range(4):
        outs[q][...] = _from64(o, q)


_dense2 = pl.pallas_call(
    _dense2_body,
    grid=(_GRID,),
    in_specs=[_packspec()] * 5 + [
              pl.BlockSpec((64, 64), lambda i: (0, 0)),
              pl.BlockSpec((1, 512), lambda i: (0, 0))],
    out_specs=[_packspec()] * 4,
    out_shape=_chunk_out,
)


def _dense3_body(t0, t1, t2, t3, dv, wc, b2p, bc, out):
    d64 = _dinv64(dv[...])
    h = jnp.maximum(_to64([t0[...], t1[...], t2[...], t3[...]]) * d64
                    + b2p[...], 0.0)
    o = jnp.concatenate(
        [jnp.dot(h[:, 64 * k:64 * k + 64], wc[...],
                 preferred_element_type=jnp.float32) for k in range(8)],
        axis=1)
    out[...] = o + bc[0, 0]


_dense3 = pl.pallas_call(
    _dense3_body,
    grid=(_GRID,),
    in_specs=[_packspec()] * 5 + [
              pl.BlockSpec((64, 1), lambda i: (0, 0)),
              pl.BlockSpec((1, 512), lambda i: (0, 0)),
              pl.BlockSpec((1, 1), lambda i: (0, 0))],
    out_specs=pl.BlockSpec((_PBLK, 8), lambda i: (i, 0)),
    out_shape=jax.ShapeDtypeStruct((PK, 8), jnp.float32),
)


def _to16(a):
    return jnp.reshape(a, (NP, 16))


def _topack(a):
    return jnp.reshape(a, (PK, 128))


def kernel(x, edge_index, W1, b1, W2, b2, Wc, bc):
    src = edge_index[0].astype(jnp.int32)
    dst = edge_index[1].astype(jnp.int32)
    pad = EP - E
    src1d = jnp.concatenate([src, jnp.zeros((pad,), jnp.int32)])
    dst1d = jnp.concatenate([dst, jnp.full((pad,), N, jnp.int32)])
    zeros16 = jnp.zeros((NP, 16), jnp.float32)
    ones16 = jnp.ones((SCE, 16), jnp.float32)

    b1p = jnp.tile(b1, 8).reshape(1, 512)
    b2p = jnp.tile(b2, 8).reshape(1, 512)
    xp = x.reshape(N // 8, 1024)

    d0, d1 = _deg_kernel(dst1d, zeros16, ones16)
    dp0, dp1 = _topack(d0), _topack(d1)
    *g, dv = _dense1(dp0, dp1, xp, W1)
    s = _scatter_kernel(src1d, dst1d, *[_to16(a) for a in g])
    g2 = _dense2(*[_topack(a) for a in s], dv, W2, b1p)
    t = _scatter_kernel(src1d, dst1d, *[_to16(a) for a in g2])
    out = _dense3(*[_topack(a) for a in t], dv, Wc, b2p,
                  bc.reshape(1, 1))
    return out.reshape(NP)[:N]


# speedup vs baseline: 1.0234x; 1.0234x over previous
import functools

import jax
import jax.numpy as jnp
from jax import lax
from jax.experimental import pallas as pl
from jax.experimental.pallas import tpu as pltpu
from jax.experimental.pallas import tpu_sc as plsc

N = 50000
E = 800000
NP = 50176
EP = 802816
ER2 = EP // 128
RPT = NP // 16
CONV_ROWS_PT = ER2 // 16
DEG_ROWS_PT = ER2 // 32
PK = NP // 8
SCE = 896
EPT = EP // 16
NB = EPT // (2 * SCE)
DEPC = EP // 32
DNB = DEPC // (2 * SCE)

_mesh = plsc.VectorSubcoreMesh(core_axis_name="c", subcore_axis_name="s")
_sc_params = pltpu.CompilerParams(use_tc_tiling_on_sc=False)


@functools.partial(
    pl.kernel,
    mesh=_mesh,
    compiler_params=_sc_params,
    out_type=(jax.ShapeDtypeStruct((NP, 16), jnp.float32),
              jax.ShapeDtypeStruct((NP, 16), jnp.float32)),
    scratch_types=[
        pltpu.VMEM((SCE,), jnp.int32),
        pltpu.VMEM((SCE,), jnp.int32),
        pltpu.VMEM((SCE, 16), jnp.float32),
        pltpu.VMEM((RPT, 16), jnp.float32),
        pltpu.VMEM_SHARED((NP, 16), jnp.float32),
        pltpu.SemaphoreType.DMA,
        pltpu.SemaphoreType.DMA,
    ],
)
def _deg_kernel(dst1d, zeros_hbm, ones_hbm, deg0, deg1,
                didxa, didxb, ones_v, stage, acc, sema, semb):
    c = lax.axis_index("c")
    s = lax.axis_index("s")
    pltpu.sync_copy(ones_hbm, ones_v)
    pltpu.sync_copy(zeros_hbm.at[pl.ds(s * RPT, RPT)],
                    acc.at[pl.ds(s * RPT, RPT)])
    plsc.subcore_barrier()
    base = (c * 16 + s) * DEPC

    def drain(sem):
        pltpu.make_async_copy(zeros_hbm.at[pl.ds(0, SCE)], ones_v, sem).wait()

    pltpu.sync_copy(dst1d.at[pl.ds(base, SCE)], didxa)

    def body(i, carry):
        pltpu.async_copy(ones_v, acc.at[didxa], sema, add=True)
        pltpu.sync_copy(dst1d.at[pl.ds(base + (2 * i + 1) * SCE, SCE)], didxb)
        drain(sema)
        pltpu.async_copy(ones_v, acc.at[didxb], semb, add=True)
        rb2 = jnp.minimum(base + (2 * i + 2) * SCE,
                          base + (2 * DNB - 1) * SCE)
        pltpu.sync_copy(dst1d.at[pl.ds(rb2, SCE)], didxa)
        drain(semb)
        return carry

    lax.fori_loop(0, DNB, body, 0)
    plsc.subcore_barrier()

    @pl.when(c == 0)
    def _():
        pltpu.sync_copy(acc.at[pl.ds(s * RPT, RPT)],
                        deg0.at[pl.ds(s * RPT, RPT)])

    @pl.when(c == 1)
    def _():
        pltpu.sync_copy(acc.at[pl.ds(s * RPT, RPT)],
                        deg1.at[pl.ds(s * RPT, RPT)])


@functools.partial(
    pl.kernel,
    mesh=_mesh,
    compiler_params=_sc_params,
    out_type=tuple(jax.ShapeDtypeStruct((NP, 16), jnp.float32)
                   for _ in range(4)),
    scratch_types=[
        pltpu.VMEM((SCE,), jnp.int32),
        pltpu.VMEM((SCE,), jnp.int32),
        pltpu.VMEM((SCE,), jnp.int32),
        pltpu.VMEM((SCE,), jnp.int32),
        pltpu.VMEM((SCE, 16), jnp.float32),
        pltpu.VMEM((SCE, 16), jnp.float32),
        pltpu.VMEM_SHARED((NP, 16), jnp.float32),
        pltpu.SemaphoreType.DMA,
        pltpu.SemaphoreType.DMA,
        pltpu.SemaphoreType.DMA,
        pltpu.SemaphoreType.DMA,
    ],
)
def _scatter_kernel(src1d, dst1d, g0, g1, g2, g3, o0, o1, o2, o3,
                    sidx0, sidx1, didx0, didx1, rows0, rows1, acc,
                    gsem0, gsem1, ssem0, ssem1):
    c = lax.axis_index("c")
    s = lax.axis_index("s")

    def run(g_hbm, out_hbm):
        pltpu.sync_copy(g_hbm.at[pl.ds(s * RPT, RPT)],
                        acc.at[pl.ds(s * RPT, RPT)])
        plsc.subcore_barrier()

        base = s * EPT

        def load_idx(eb, sidx, didx):
            pltpu.sync_copy(src1d.at[pl.ds(eb, SCE)], sidx)
            pltpu.sync_copy(dst1d.at[pl.ds(eb, SCE)], didx)

        def drain(sem):
            pltpu.make_async_copy(g_hbm.at[pl.ds(0, SCE)], rows0, sem).wait()

        load_idx(base, sidx0, didx0)
        pltpu.async_copy(g_hbm.at[sidx0], rows0, gsem0)

        def body(i, carry):
            load_idx(base + (2 * i + 1) * SCE, sidx1, didx1)
            drain(gsem0)
            pltpu.async_copy(rows0, acc.at[didx0], ssem0, add=True)
            pltpu.async_copy(g_hbm.at[sidx1], rows1, gsem1)
            drain(ssem0)
            eb2 = jnp.minimum(base + (2 * i + 2) * SCE,
                              base + (2 * NB - 1) * SCE)
            load_idx(eb2, sidx0, didx0)
            drain(gsem1)
            pltpu.async_copy(rows1, acc.at[didx1], ssem1, add=True)

            @pl.when(i < NB - 1)
            def _():
                pltpu.async_copy(g_hbm.at[sidx0], rows0, gsem0)
            drain(ssem1)
            return carry

        lax.fori_loop(0, NB, body, 0)
        plsc.subcore_barrier()
        pltpu.sync_copy(acc.at[pl.ds(s * RPT, RPT)],
                        out_hbm.at[pl.ds(s * RPT, RPT)])

    gs = (g0, g1, g2, g3)
    outs = (o0, o1, o2, o3)
    for q in range(4):
        @pl.when(c == q // 2)
        def _(q=q):
            run(gs[q], outs[q])


_PBLK = 128
_GRID = PK // _PBLK


def _packspec():
    return pl.BlockSpec((_PBLK, 128), lambda i: (i, 0))


def _dinv_pk(d0_ref, d1_ref):
    return lax.rsqrt(d0_ref[...] + d1_ref[...] + 1.0)


def _dinv64(dpk):
    return jnp.concatenate(
        [jnp.broadcast_to(dpk[:, 16 * k:16 * k + 1], (_PBLK, 64))
         for k in range(8)], axis=1)


def _to64(chunks):
    cols = []
    for k in range(8):
        for q in range(4):
            cols.append(chunks[q][:, 16 * k:16 * k + 16])
    return jnp.concatenate(cols, axis=1)


def _from64(o, q):
    return jnp.concatenate(
        [o[:, 64 * k + 16 * q:64 * k + 16 * q + 16] for k in range(8)],
        axis=1)


def _dense1_body(d0, d1, xp, w1, g0, g1, g2, g3, dvo):
    d64 = _dinv64(_dinv_pk(d0, d1))
    xb = xp[...]
    o = jnp.concatenate(
        [jnp.dot(xb[:, 128 * k:128 * k + 128], w1[...],
                 preferred_element_type=jnp.float32) for k in range(8)],
        axis=1) * d64
    outs = (g0, g1, g2, g3)
    for q in range(4):
        outs[q][...] = _from64(o, q)
    dvo[...] = _dinv_pk(d0, d1)


_chunk_out = tuple(jax.ShapeDtypeStruct((PK, 128), jnp.float32)
                   for _ in range(4))

_dense1 = pl.pallas_call(
    _dense1_body,
    grid=(_GRID,),
    in_specs=[_packspec(), _packspec(),
              pl.BlockSpec((_PBLK, 1024), lambda i: (i, 0)),
              pl.BlockSpec((128, 64), lambda i: (0, 0))],
    out_specs=[_packspec()] * 5,
    out_shape=_chunk_out + (jax.ShapeDtypeStruct((PK, 128), jnp.float32),),
)


def _dense2_body(s0, s1, s2, s3, dv, w2, b1p, g20, g21, g22, g23):
    d64 = _dinv64(dv[...])
    h = jnp.maximum(_to64([s0[...], s1[...], s2[...], s3[...]]) * d64
                    + b1p[...], 0.0)
    o = jnp.concatenate(
        [jnp.dot(h[:, 64 * k:64 * k + 64], w2[...],
                 preferred_element_type=jnp.float32) for k in range(8)],
        axis=1) * d64
    outs = (g20, g21, g22, g23)
    for q in range(4):
        outs[q][...] = _from64(o, q)


_dense2 = pl.pallas_call(
    _dense2_body,
    grid=(_GRID,),
    in_specs=[_packspec()] * 5 + [
              pl.BlockSpec((64, 64), lambda i: (0, 0)),
              pl.BlockSpec((1, 512), lambda i: (0, 0))],
    out_specs=[_packspec()] * 4,
    out_shape=_chunk_out,
)


def _dense3_body(t0, t1, t2, t3, dv, wc, b2p, bc, out):
    d64 = _dinv64(dv[...])
    h = jnp.maximum(_to64([t0[...], t1[...], t2[...], t3[...]]) * d64
                    + b2p[...], 0.0)
    o = jnp.concatenate(
        [jnp.dot(h[:, 64 * k:64 * k + 64], wc[...],
                 preferred_element_type=jnp.float32) for k in range(8)],
        axis=1)
    out[...] = o + bc[0, 0]


_dense3 = pl.pallas_call(
    _dense3_body,
    grid=(_GRID,),
    in_specs=[_packspec()] * 5 + [
              pl.BlockSpec((64, 1), lambda i: (0, 0)),
              pl.BlockSpec((1, 512), lambda i: (0, 0)),
              pl.BlockSpec((1, 1), lambda i: (0, 0))],
    out_specs=pl.BlockSpec((_PBLK, 8), lambda i: (i, 0)),
    out_shape=jax.ShapeDtypeStruct((PK, 8), jnp.float32),
)


def _to16(a):
    return jnp.reshape(a, (NP, 16))


def _topack(a):
    return jnp.reshape(a, (PK, 128))


def kernel(x, edge_index, W1, b1, W2, b2, Wc, bc):
    src = edge_index[0].astype(jnp.int32)
    dst = edge_index[1].astype(jnp.int32)
    pad = EP - E
    src1d = jnp.concatenate([src, jnp.zeros((pad,), jnp.int32)])
    dst1d = jnp.concatenate([dst, jnp.full((pad,), N, jnp.int32)])
    zeros16 = jnp.zeros((NP, 16), jnp.float32)
    ones16 = jnp.ones((SCE, 16), jnp.float32)

    b1p = jnp.tile(b1, 8).reshape(1, 512)
    b2p = jnp.tile(b2, 8).reshape(1, 512)
    xp = x.reshape(N // 8, 1024)

    d0, d1 = _deg_kernel(dst1d, zeros16, ones16)
    dp0, dp1 = _topack(d0), _topack(d1)
    *g, dv = _dense1(dp0, dp1, xp, W1)
    s = _scatter_kernel(src1d, dst1d, *[_to16(a) for a in g])
    g2 = _dense2(*[_topack(a) for a in s], dv, W2, b1p)
    t = _scatter_kernel(src1d, dst1d, *[_to16(a) for a in g2])
    out = _dense3(*[_topack(a) for a in t], dv, Wc, b2p,
                  bc.reshape(1, 1))
    return out.reshape(NP)[:N]


# speedup vs baseline: 1.1182x; 1.0926x over previous
import functools

import jax
import jax.numpy as jnp
from jax import lax
from jax.experimental import pallas as pl
from jax.experimental.pallas import tpu as pltpu
from jax.experimental.pallas import tpu_sc as plsc

N = 50000
E = 800000
NP = 50176
EP = 802816
ER2 = EP // 128
RPT = NP // 16
CONV_ROWS_PT = ER2 // 16
DEG_ROWS_PT = ER2 // 32
PK = NP // 8
SCE = 1568
EPT = EP // 16
NB = EPT // (2 * SCE)
DEPC = EP // 32
DNB = DEPC // (2 * SCE)

_mesh = plsc.VectorSubcoreMesh(core_axis_name="c", subcore_axis_name="s")
_sc_params = pltpu.CompilerParams(use_tc_tiling_on_sc=False)


@functools.partial(
    pl.kernel,
    mesh=_mesh,
    compiler_params=_sc_params,
    out_type=(jax.ShapeDtypeStruct((NP, 16), jnp.float32),
              jax.ShapeDtypeStruct((NP, 16), jnp.float32)),
    scratch_types=[
        pltpu.VMEM((SCE,), jnp.int32),
        pltpu.VMEM((SCE,), jnp.int32),
        pltpu.VMEM((SCE, 16), jnp.float32),
        pltpu.VMEM((RPT, 16), jnp.float32),
        pltpu.VMEM_SHARED((NP, 16), jnp.float32),
        pltpu.SemaphoreType.DMA,
        pltpu.SemaphoreType.DMA,
    ],
)
def _deg_kernel(dst1d, zeros_hbm, ones_hbm, deg0, deg1,
                didxa, didxb, ones_v, stage, acc, sema, semb):
    c = lax.axis_index("c")
    s = lax.axis_index("s")
    pltpu.sync_copy(ones_hbm, ones_v)
    pltpu.sync_copy(zeros_hbm.at[pl.ds(s * RPT, RPT)],
                    acc.at[pl.ds(s * RPT, RPT)])
    plsc.subcore_barrier()
    base = (c * 16 + s) * DEPC

    def drain(sem):
        pltpu.make_async_copy(zeros_hbm.at[pl.ds(0, SCE)], ones_v, sem).wait()

    pltpu.sync_copy(dst1d.at[pl.ds(base, SCE)], didxa)

    def body(i, carry):
        pltpu.async_copy(ones_v, acc.at[didxa], sema, add=True)
        pltpu.sync_copy(dst1d.at[pl.ds(base + (2 * i + 1) * SCE, SCE)], didxb)
        drain(sema)
        pltpu.async_copy(ones_v, acc.at[didxb], semb, add=True)
        rb2 = jnp.minimum(base + (2 * i + 2) * SCE,
                          base + (2 * DNB - 1) * SCE)
        pltpu.sync_copy(dst1d.at[pl.ds(rb2, SCE)], didxa)
        drain(semb)
        return carry

    lax.fori_loop(0, DNB, body, 0)
    plsc.subcore_barrier()

    @pl.when(c == 0)
    def _():
        pltpu.sync_copy(acc.at[pl.ds(s * RPT, RPT)],
                        deg0.at[pl.ds(s * RPT, RPT)])

    @pl.when(c == 1)
    def _():
        pltpu.sync_copy(acc.at[pl.ds(s * RPT, RPT)],
                        deg1.at[pl.ds(s * RPT, RPT)])


@functools.partial(
    pl.kernel,
    mesh=_mesh,
    compiler_params=_sc_params,
    out_type=tuple(jax.ShapeDtypeStruct((NP, 16), jnp.float32)
                   for _ in range(4)),
    scratch_types=[
        pltpu.VMEM((SCE,), jnp.int32),
        pltpu.VMEM((SCE,), jnp.int32),
        pltpu.VMEM((SCE,), jnp.int32),
        pltpu.VMEM((SCE,), jnp.int32),
        pltpu.VMEM((SCE, 16), jnp.float32),
        pltpu.VMEM((SCE, 16), jnp.float32),
        pltpu.VMEM_SHARED((NP, 16), jnp.float32),
        pltpu.SemaphoreType.DMA,
        pltpu.SemaphoreType.DMA,
        pltpu.SemaphoreType.DMA,
        pltpu.SemaphoreType.DMA,
    ],
)
def _scatter_kernel(src1d, dst1d, g0, g1, g2, g3, o0, o1, o2, o3,
                    sidx0, sidx1, didx0, didx1, rows0, rows1, acc,
                    gsem0, gsem1, ssem0, ssem1):
    c = lax.axis_index("c")
    s = lax.axis_index("s")

    def run(g_hbm, out_hbm):
        pltpu.sync_copy(g_hbm.at[pl.ds(s * RPT, RPT)],
                        acc.at[pl.ds(s * RPT, RPT)])
        plsc.subcore_barrier()

        base = s * EPT

        def load_idx(eb, sidx, didx):
            pltpu.sync_copy(src1d.at[pl.ds(eb, SCE)], sidx)
            pltpu.sync_copy(dst1d.at[pl.ds(eb, SCE)], didx)

        def drain(sem):
            pltpu.make_async_copy(g_hbm.at[pl.ds(0, SCE)], rows0, sem).wait()

        load_idx(base, sidx0, didx0)
        pltpu.async_copy(g_hbm.at[sidx0], rows0, gsem0)

        def body(i, carry):
            load_idx(base + (2 * i + 1) * SCE, sidx1, didx1)
            drain(gsem0)
            pltpu.async_copy(rows0, acc.at[didx0], ssem0, add=True)
            pltpu.async_copy(g_hbm.at[sidx1], rows1, gsem1)
            drain(ssem0)
            eb2 = jnp.minimum(base + (2 * i + 2) * SCE,
                              base + (2 * NB - 1) * SCE)
            load_idx(eb2, sidx0, didx0)
            drain(gsem1)
            pltpu.async_copy(rows1, acc.at[didx1], ssem1, add=True)

            @pl.when(i < NB - 1)
            def _():
                pltpu.async_copy(g_hbm.at[sidx0], rows0, gsem0)
            drain(ssem1)
            return carry

        lax.fori_loop(0, NB, body, 0)
        plsc.subcore_barrier()
        pltpu.sync_copy(acc.at[pl.ds(s * RPT, RPT)],
                        out_hbm.at[pl.ds(s * RPT, RPT)])

    gs = (g0, g1, g2, g3)
    outs = (o0, o1, o2, o3)
    for q in range(4):
        @pl.when(c == q // 2)
        def _(q=q):
            run(gs[q], outs[q])


_PBLK = 128
_GRID = PK // _PBLK


def _packspec():
    return pl.BlockSpec((_PBLK, 128), lambda i: (i, 0))


def _dinv_pk(d0_ref, d1_ref):
    return lax.rsqrt(d0_ref[...] + d1_ref[...] + 1.0)


def _dinv64(dpk):
    return jnp.concatenate(
        [jnp.broadcast_to(dpk[:, 16 * k:16 * k + 1], (_PBLK, 64))
         for k in range(8)], axis=1)


def _to64(chunks):
    cols = []
    for k in range(8):
        for q in range(4):
            cols.append(chunks[q][:, 16 * k:16 * k + 16])
    return jnp.concatenate(cols, axis=1)


def _from64(o, q):
    return jnp.concatenate(
        [o[:, 64 * k + 16 * q:64 * k + 16 * q + 16] for k in range(8)],
        axis=1)


def _dense1_body(d0, d1, xp, w1, g0, g1, g2, g3, dvo):
    d64 = _dinv64(_dinv_pk(d0, d1))
    xb = xp[...]
    o = jnp.concatenate(
        [jnp.dot(xb[:, 128 * k:128 * k + 128], w1[...],
                 preferred_element_type=jnp.float32) for k in range(8)],
        axis=1) * d64
    outs = (g0, g1, g2, g3)
    for q in range(4):
        outs[q][...] = _from64(o, q)
    dvo[...] = _dinv_pk(d0, d1)


_chunk_out = tuple(jax.ShapeDtypeStruct((PK, 128), jnp.float32)
                   for _ in range(4))

_dense1 = pl.pallas_call(
    _dense1_body,
    grid=(_GRID,),
    in_specs=[_packspec(), _packspec(),
              pl.BlockSpec((_PBLK, 1024), lambda i: (i, 0)),
              pl.BlockSpec((128, 64), lambda i: (0, 0))],
    out_specs=[_packspec()] * 5,
    out_shape=_chunk_out + (jax.ShapeDtypeStruct((PK, 128), jnp.float32),),
)


def _dense2_body(s0, s1, s2, s3, dv, w2, b1p, g20, g21, g22, g23):
    d64 = _dinv64(dv[...])
    h = jnp.maximum(_to64([s0[...], s1[...], s2[...], s3[...]]) * d64
                    + b1p[...], 0.0)
    o = jnp.concatenate(
        [jnp.dot(h[:, 64 * k:64 * k + 64], w2[...],
                 preferred_element_type=jnp.float32) for k in range(8)],
        axis=1) * d64
    outs = (g20, g21, g22, g23)
    for q in range(4):
        outs[q][...] = _from64(o, q)


_dense2 = pl.pallas_call(
    _dense2_body,
    grid=(_GRID,),
    in_specs=[_packspec()] * 5 + [
              pl.BlockSpec((64, 64), lambda i: (0, 0)),
              pl.BlockSpec((1, 512), lambda i: (0, 0))],
    out_specs=[_packspec()] * 4,
    out_shape=_chunk_out,
)


def _dense3_body(t0, t1, t2, t3, dv, wc, b2p, bc, out):
    d64 = _dinv64(dv[...])
    h = jnp.maximum(_to64([t0[...], t1[...], t2[...], t3[...]]) * d64
                    + b2p[...], 0.0)
    o = jnp.concatenate(
        [jnp.dot(h[:, 64 * k:64 * k + 64], wc[...],
                 preferred_element_type=jnp.float32) for k in range(8)],
        axis=1)
    out[...] = o + bc[0, 0]


_dense3 = pl.pallas_call(
    _dense3_body,
    grid=(_GRID,),
    in_specs=[_packspec()] * 5 + [
              pl.BlockSpec((64, 1), lambda i: (0, 0)),
              pl.BlockSpec((1, 512), lambda i: (0, 0)),
              pl.BlockSpec((1, 1), lambda i: (0, 0))],
    out_specs=pl.BlockSpec((_PBLK, 8), lambda i: (i, 0)),
    out_shape=jax.ShapeDtypeStruct((PK, 8), jnp.float32),
)


def _to16(a):
    return jnp.reshape(a, (NP, 16))


def _topack(a):
    return jnp.reshape(a, (PK, 128))


def kernel(x, edge_index, W1, b1, W2, b2, Wc, bc):
    src = edge_index[0].astype(jnp.int32)
    dst = edge_index[1].astype(jnp.int32)
    pad = EP - E
    src1d = jnp.concatenate([src, jnp.zeros((pad,), jnp.int32)])
    dst1d = jnp.concatenate([dst, jnp.full((pad,), N, jnp.int32)])
    zeros16 = jnp.zeros((NP, 16), jnp.float32)
    ones16 = jnp.ones((SCE, 16), jnp.float32)

    b1p = jnp.tile(b1, 8).reshape(1, 512)
    b2p = jnp.tile(b2, 8).reshape(1, 512)
    xp = x.reshape(N // 8, 1024)

    d0, d1 = _deg_kernel(dst1d, zeros16, ones16)
    dp0, dp1 = _topack(d0), _topack(d1)
    *g, dv = _dense1(dp0, dp1, xp, W1)
    s = _scatter_kernel(src1d, dst1d, *[_to16(a) for a in g])
    g2 = _dense2(*[_topack(a) for a in s], dv, W2, b1p)
    t = _scatter_kernel(src1d, dst1d, *[_to16(a) for a in g2])
    out = _dense3(*[_topack(a) for a in t], dv, Wc, b2p,
                  bc.reshape(1, 1))
    return out.reshape(NP)[:N]
